# baseline (speedup 1.0000x reference)
import jax
import jax.numpy as jnp
from jax import lax
from jax.experimental import pallas as pl
from jax.experimental.pallas import tpu as pltpu

_BI = 256


def _cast1_body(w_ref, abf_ref, u_ref, vp_ref):
    i = pl.program_id(0)

    @pl.when(i == 0)
    def _():
        vp_ref[...] = jnp.zeros_like(vp_ref)

    a = jnp.abs(w_ref[...])
    abf_ref[...] = a.astype(jnp.bfloat16)
    u_blk = 1.0 / jnp.sum(a, axis=1, keepdims=True)
    u_ref[...] = u_blk
    vp_ref[...] += jnp.sum(a * u_blk, axis=0, keepdims=True)


def _iter_body(abf_ref, vpin_ref, u_ref, vp_ref, v_scr):
    i = pl.program_id(0)

    @pl.when(i == 0)
    def _():
        v_scr[...] = 1.0 / vpin_ref[...]
        vp_ref[...] = jnp.zeros_like(vp_ref)

    a = abf_ref[...].astype(jnp.float32)
    s = jnp.sum(a * v_scr[...], axis=1, keepdims=True)
    u_blk = 1.0 / s
    u_ref[...] = u_blk
    vp_ref[...] += jnp.sum(a * u_blk, axis=0, keepdims=True)


def _final_body(abf_ref, u_ref, vpin_ref, out_ref, v_scr):
    i = pl.program_id(0)

    @pl.when(i == 0)
    def _():
        v_scr[...] = 1.0 / vpin_ref[...]

    out_ref[...] = abf_ref[...].astype(jnp.float32) * u_ref[...] * v_scr[...]


def kernel(W):
    n = W.shape[0]
    bi = min(_BI, n)
    nb = n // bi
    grid = (nb,)
    params = pltpu.CompilerParams(dimension_semantics=("arbitrary",))

    blk_mat = pl.BlockSpec((bi, n), lambda i: (i, 0))
    blk_u = pl.BlockSpec((bi, 1), lambda i: (i, 0))
    blk_vp = pl.BlockSpec((1, n), lambda i: (0, 0))

    abf, u, vp = pl.pallas_call(
        _cast1_body,
        grid=grid,
        in_specs=[blk_mat],
        out_specs=[blk_mat, blk_u, blk_vp],
        out_shape=[
            jax.ShapeDtypeStruct((n, n), jnp.bfloat16),
            jax.ShapeDtypeStruct((n, 1), jnp.float32),
            jax.ShapeDtypeStruct((1, n), jnp.float32),
        ],
        compiler_params=params,
    )(W)

    iter_call = pl.pallas_call(
        _iter_body,
        grid=grid,
        in_specs=[blk_mat, blk_vp],
        out_specs=[blk_u, blk_vp],
        out_shape=[
            jax.ShapeDtypeStruct((n, 1), jnp.float32),
            jax.ShapeDtypeStruct((1, n), jnp.float32),
        ],
        scratch_shapes=[pltpu.VMEM((1, n), jnp.float32)],
        compiler_params=params,
    )

    u, vp = lax.fori_loop(
        0, 9, lambda _, c: iter_call(abf, c[1]), (u, vp))

    out = pl.pallas_call(
        _final_body,
        grid=grid,
        in_specs=[blk_mat, blk_u, blk_vp],
        out_specs=blk_mat,
        out_shape=jax.ShapeDtypeStruct((n, n), jnp.float32),
        scratch_shapes=[pltpu.VMEM((1, n), jnp.float32)],
        compiler_params=params,
    )(abf, u, vp)
    return out


# speedup vs baseline: 1.0202x; 1.0202x over previous
import jax
import jax.numpy as jnp
from jax import lax
from jax.experimental import pallas as pl
from jax.experimental.pallas import tpu as pltpu

_BI = 256
_BI_ITER = 256


def _cast1_body(w_ref, abf_ref, u_ref, vp_ref):
    i = pl.program_id(0)

    @pl.when(i == 0)
    def _():
        vp_ref[...] = jnp.zeros_like(vp_ref)

    a = jnp.abs(w_ref[...])
    abf_ref[...] = a.astype(jnp.bfloat16)
    u_blk = 1.0 / jnp.sum(a, axis=1, keepdims=True)
    u_ref[...] = u_blk
    vp_ref[...] += jnp.sum(a * u_blk, axis=0, keepdims=True)


def _iter_body(abf_ref, vpin_ref, u_ref, vp_ref, v_scr):
    i = pl.program_id(0)

    @pl.when(i == 0)
    def _():
        v_scr[...] = 1.0 / vpin_ref[...]
        vp_ref[...] = jnp.zeros_like(vp_ref)

    a = abf_ref[...].astype(jnp.float32)
    s = jnp.sum(a * v_scr[...], axis=1, keepdims=True)
    u_blk = 1.0 / s
    u_ref[...] = u_blk
    vp_ref[...] += jnp.sum(a * u_blk, axis=0, keepdims=True)


def _final_body(abf_ref, u_ref, vpin_ref, out_ref, v_scr):
    i = pl.program_id(0)

    @pl.when(i == 0)
    def _():
        v_scr[...] = 1.0 / vpin_ref[...]

    out_ref[...] = abf_ref[...].astype(jnp.float32) * u_ref[...] * v_scr[...]


def kernel(W):
    n = W.shape[0]
    bi = min(_BI_ITER, n)
    bs = min(_BI, n)
    nb = n // bi
    grid = (nb,)
    grid_s = (n // bs,)
    params = pltpu.CompilerParams(dimension_semantics=("arbitrary",))

    blk_mat = pl.BlockSpec((bi, n), lambda i: (i, 0))
    blk_u = pl.BlockSpec((bi, 1), lambda i: (i, 0))
    blk_mat_s = pl.BlockSpec((bs, n), lambda i: (i, 0))
    blk_u_s = pl.BlockSpec((bs, 1), lambda i: (i, 0))
    blk_vp = pl.BlockSpec((1, n), lambda i: (0, 0))

    abf, u, vp = pl.pallas_call(
        _cast1_body,
        grid=grid_s,
        in_specs=[blk_mat_s],
        out_specs=[blk_mat_s, blk_u_s, blk_vp],
        out_shape=[
            jax.ShapeDtypeStruct((n, n), jnp.bfloat16),
            jax.ShapeDtypeStruct((n, 1), jnp.float32),
            jax.ShapeDtypeStruct((1, n), jnp.float32),
        ],
        compiler_params=params,
    )(W)

    iter_call = pl.pallas_call(
        _iter_body,
        grid=grid,
        in_specs=[blk_mat, blk_vp],
        out_specs=[blk_u, blk_vp],
        out_shape=[
            jax.ShapeDtypeStruct((n, 1), jnp.float32),
            jax.ShapeDtypeStruct((1, n), jnp.float32),
        ],
        scratch_shapes=[pltpu.VMEM((1, n), jnp.float32)],
        compiler_params=params,
    )

    u, vp = lax.fori_loop(
        0, 9, lambda _, c: iter_call(abf, c[1]), (u, vp))

    out = pl.pallas_call(
        _final_body,
        grid=grid_s,
        in_specs=[blk_mat_s, blk_u_s, blk_vp],
        out_specs=blk_mat_s,
        out_shape=jax.ShapeDtypeStruct((n, n), jnp.float32),
        scratch_shapes=[pltpu.VMEM((1, n), jnp.float32)],
        compiler_params=params,
    )(abf, u, vp)
    return out
